# Initial kernel scaffold; baseline (speedup 1.0000x reference)
import jax
import jax.numpy as jnp
from jax.experimental import pallas as pl


def kernel(users, items, user_emb, item_emb, edge_src, edge_dst, edge_val):
    raise NotImplementedError("write your pallas kernel here")


# speedup vs baseline: 13.5771x; 13.5771x over previous
import functools

import jax
import jax.numpy as jnp
from jax import lax
from jax.experimental import pallas as pl
from jax.experimental.pallas import tpu as pltpu
from jax.experimental.pallas import tpu_sc as plsc

_NU = 25000
_NI = 25000
_N = _NU + _NI
_NP = 50048
_H = 16
_E = 1600000
_CH = 1024
_SUB = 128
_NSUB = _CH // _SUB
_KCH = 98
_EPAD = 16 * _KCH * _CH
_ROWS_PER_TILE = _NP // 16
_POST = [(0, 1024), (1024, 1024), (2048, 1024), (3072, 56)]
_BATCH = 4096


@functools.partial(
    pl.kernel,
    out_type=[
        jax.ShapeDtypeStruct((2, _BATCH, _H), jnp.float32),
        jax.ShapeDtypeStruct((2, _BATCH, _H), jnp.float32),
        jax.ShapeDtypeStruct((2 * _NP, _H), jnp.float32),
        jax.ShapeDtypeStruct((2 * _NP, _H), jnp.float32),
        jax.ShapeDtypeStruct((2 * _NP, _H), jnp.float32),
    ],
    mesh=plsc.VectorSubcoreMesh(core_axis_name="c", subcore_axis_name="s"),
    compiler_params=pltpu.CompilerParams(use_tc_tiling_on_sc=False),
    scratch_types=[
        pltpu.VMEM((_CH, _H), jnp.float32),
        pltpu.VMEM((_CH, _H), jnp.float32),
        pltpu.VMEM((_CH, _H), jnp.float32),
        pltpu.VMEM((_NSUB, _SUB), jnp.int32),
        pltpu.VMEM((_NSUB, _SUB), jnp.int32),
        pltpu.VMEM((_NSUB, _SUB), jnp.float32),
        pltpu.VMEM_SHARED((_NP, _H), jnp.float32),
        pltpu.SemaphoreType.DMA,
    ],
)
def _lightgcn_sc(t0_r, src2_r, dst3_r, val2_r, users2_r, items2_r,
                 suo_r, sio_r, ta_r, tb_r, sum_r,
                 rows0, rows1, zbuf, src_v, dst_v, val_v, acc, sem):
    c = lax.axis_index("c")
    s = lax.axis_index("s")
    zero16 = jnp.zeros((_H,), jnp.float32)

    def _zb(i, carry):
        zbuf[i, :] = zero16
        return carry

    lax.fori_loop(0, _CH, _zb, 0)

    base = s * _ROWS_PER_TILE
    for off, sz in _POST:
        pltpu.sync_copy(zbuf.at[pl.ds(0, sz)], acc.at[pl.ds(base + off, sz)])
    plsc.subcore_barrier()

    def edge_pass(tbl_r):
        def chunk_body(k, carry):
            cid = s * _KCH + k
            pltpu.sync_copy(src2_r.at[c, cid], src_v)
            pltpu.sync_copy(dst3_r.at[cid], dst_v)
            pltpu.sync_copy(val2_r.at[cid], val_v)
            handles = [
                pltpu.async_copy(tbl_r.at[src_v.at[j]],
                                 rows0.at[pl.ds(j * _SUB, _SUB)], sem)
                for j in range(_NSUB)
            ]
            for h in handles:
                h.wait()

            for j in range(_NSUB):
                def scale(m, cc, _j=j):
                    b0 = _j * _SUB + m * 16
                    vv = val_v[_j, pl.ds(m * 16, 16)]
                    for t in range(16):
                        rows0[b0 + t, :] = rows0[b0 + t, :] * vv[t]
                    return cc

                lax.fori_loop(0, _SUB // 16, scale, 0)
            for j in range(_NSUB):
                pltpu.sync_copy(rows0.at[pl.ds(j * _SUB, _SUB)],
                                acc.at[dst_v.at[j]], add=True)
            return carry

        lax.fori_loop(0, _KCH, chunk_body, 0)

    def post_pass(sum_src_r, tnext_r):
        for off, sz in _POST:
            r0 = rows0.at[pl.ds(0, sz)]
            hrow = c * _NP + base + off
            pltpu.sync_copy(acc.at[pl.ds(base + off, sz)], r0)
            if tnext_r is not None:
                pltpu.sync_copy(r0, tnext_r.at[pl.ds(hrow, sz)])
            pltpu.sync_copy(sum_src_r.at[pl.ds(hrow, sz)], rows1.at[pl.ds(0, sz)])

            def addb(i, cc):
                rows0[i, :] = rows0[i, :] + rows1[i, :]
                return cc

            lax.fori_loop(0, sz, addb, 0)
            pltpu.sync_copy(r0, sum_r.at[pl.ds(hrow, sz)])
            pltpu.sync_copy(zbuf.at[pl.ds(0, sz)], acc.at[pl.ds(base + off, sz)])
        plsc.subcore_barrier()

    edge_pass(t0_r)
    plsc.subcore_barrier()
    post_pass(t0_r, ta_r)
    edge_pass(ta_r)
    plsc.subcore_barrier()
    post_pass(sum_r, tb_r)
    edge_pass(tb_r)
    plsc.subcore_barrier()
    post_pass(sum_r, None)

    pltpu.sync_copy(users2_r.at[c, s], src_v.at[pl.ds(0, 2)])
    pltpu.sync_copy(items2_r.at[c, s], dst_v.at[pl.ds(0, 2)])
    hs = [pltpu.async_copy(sum_r.at[src_v.at[j]],
                           rows0.at[pl.ds(j * _SUB, _SUB)], sem) for j in range(2)]
    hs += [pltpu.async_copy(sum_r.at[dst_v.at[j]],
                            rows1.at[pl.ds(j * _SUB, _SUB)], sem) for j in range(2)]
    for h in hs:
        h.wait()
    pltpu.sync_copy(rows0.at[pl.ds(0, 256)], suo_r.at[c, pl.ds(s * 256, 256)])
    pltpu.sync_copy(rows1.at[pl.ds(0, 256)], sio_r.at[c, pl.ds(s * 256, 256)])


def kernel(users, items, user_emb, item_emb, edge_src, edge_dst, edge_val):
    all0 = jnp.concatenate([user_emb, item_emb], axis=0)
    all0 = jnp.pad(all0, ((0, _NP - _N), (0, 0)))
    t0 = jnp.concatenate([all0[:, :_H], all0[:, _H:]], axis=0)
    npad = _EPAD - _E
    pad_idx = jnp.arange(npad, dtype=jnp.int32) % _N
    srcp = jnp.concatenate([edge_src, pad_idx])
    dstp = jnp.concatenate([edge_dst, pad_idx])
    valp = jnp.concatenate([edge_val, jnp.zeros((npad,), jnp.float32)])
    src2 = jnp.stack([srcp, srcp + _NP]).reshape(2, 16 * _KCH, _NSUB, _SUB)
    dst3 = dstp.reshape(16 * _KCH, _NSUB, _SUB)
    val2 = valp.reshape(16 * _KCH, _NSUB, _SUB)
    users2 = jnp.stack([users, users + _NP]).reshape(2, 16, 2, _SUB)
    items2 = jnp.stack([items + _NU, items + _NU + _NP]).reshape(2, 16, 2, _SUB)
    suo, sio, _, _, _ = _lightgcn_sc(t0, src2, dst3, val2, users2, items2)
    return _gamma_tc(suo, sio)


def _gamma_body(su_ref, si_ref, out_ref):
    p = (su_ref[0] * si_ref[0] + su_ref[1] * si_ref[1]) * 0.0625
    out_ref[:] = jnp.sum(p, axis=-1)


def _gamma_tc(suo, sio):
    return pl.pallas_call(
        _gamma_body,
        out_shape=jax.ShapeDtypeStruct((_BATCH,), jnp.float32),
    )(suo, sio)
